# Initial kernel scaffold; baseline (speedup 1.0000x reference)
import jax
import jax.numpy as jnp
from jax.experimental import pallas as pl


def kernel(vertices, faces, pix_to_face, bary_coords):
    raise NotImplementedError("write your pallas kernel here")


# speedup vs baseline: 14.4627x; 14.4627x over previous
import jax
import jax.numpy as jnp
from jax import lax
from jax.experimental import pallas as pl
from jax.experimental.pallas import tpu as pltpu
from jax.experimental.pallas import tpu_sc as plsc

B, H, W, V, F = 1, 1080, 1920, 100000, 200000
N = H * W
NC, NS = 2, 16
NW = NC * NS

F_PAD = 200704
CF = 32
S1_CHUNKS = F_PAD // (NW * CF)

CP = 128
TOTAL_CHUNKS = N // CP
S2_CHUNKS = -(-TOTAL_CHUNKS // NW)

_params = pltpu.CompilerParams(
    use_tc_tiling_on_sc=False, needs_layout_passes=False)


def _mesh():
  return plsc.VectorSubcoreMesh(core_axis_name="c", subcore_axis_name="s")


def _worker_id():
  return lax.axis_index("s") * NC + lax.axis_index("c")


def _stage1(vpad8, faces3_flat):
  def body(vpad_hbm, fidx_hbm, fattr_hbm, idx_v, vrows_v, out_v, sem):
    wid = _worker_id()
    lane = lax.iota(jnp.int32, 16)

    @pl.loop(0, S1_CHUNKS)
    def _chunk(g):
      chunk = wid * S1_CHUNKS + g
      pltpu.sync_copy(fidx_hbm.at[pl.ds(chunk * (3 * CF), 3 * CF)], idx_v)
      pltpu.async_copy(vpad_hbm.at[idx_v], vrows_v, sem).wait()
      for i in range(CF // 16):
        l = lane + (i * 16)
        l3 = l * 3
        for k in range(3):
          row = l3 + k
          for c in range(3):
            val = plsc.load_gather(vrows_v, [row, jnp.full((16,), c, jnp.int32)])
            plsc.store_scatter(out_v, [l, jnp.full((16,), 4 * k + c, jnp.int32)], val)
      pltpu.sync_copy(out_v, fattr_hbm.at[pl.ds(chunk * CF, CF)])

  return pl.kernel(
      body,
      out_type=jax.ShapeDtypeStruct((F_PAD, 16), jnp.float32),
      mesh=_mesh(),
      compiler_params=_params,
      scratch_types=[
          pltpu.VMEM((3 * CF,), jnp.int32),
          pltpu.VMEM((3 * CF, 8), jnp.float32),
          pltpu.VMEM((CF, 16), jnp.float32),
          pltpu.SemaphoreType.DMA,
      ],
  )(vpad8, faces3_flat)


def _stage2(fattr16, pix, bary_flat):
  def body(fattr_hbm, pix_hbm, bary_hbm, point_hbm, mask_hbm,
           pidx_v, gidx_v, rows_v, bary_v, pt_v, mask_v, sem):
    wid = _worker_id()
    lane = lax.iota(jnp.int32, 16)
    zero16f = jnp.zeros((16,), jnp.float32)
    one16i = jnp.full((16,), 1, jnp.int32)
    zero16i = jnp.zeros((16,), jnp.int32)

    @pl.loop(0, S2_CHUNKS)
    def _chunk(g):
      t = g * NW + wid

      @pl.when(t < TOTAL_CHUNKS)
      def _():
        base = t * CP
        pltpu.sync_copy(pix_hbm.at[pl.ds(base, CP)], pidx_v)
        for i in range(CP // 16):
          f = pidx_v[pl.ds(i * 16, 16)]
          cov = f >= 0
          gidx_v[pl.ds(i * 16, 16)] = jnp.maximum(f, 0)
          mask_v[pl.ds(i * 16, 16)] = jnp.where(cov, one16i, zero16i)
        pltpu.async_copy(fattr_hbm.at[gidx_v], rows_v, sem).wait()
        pltpu.sync_copy(bary_hbm.at[pl.ds(3 * base, 3 * CP)], bary_v)
        for i in range(CP // 16):
          l = lane + (i * 16)
          l3 = l * 3
          cov = pidx_v[pl.ds(i * 16, 16)] >= 0
          b0 = plsc.load_gather(bary_v, [l3])
          b1 = plsc.load_gather(bary_v, [l3 + 1])
          b2 = plsc.load_gather(bary_v, [l3 + 2])
          for c in range(3):
            v0 = plsc.load_gather(rows_v, [l, jnp.full((16,), c, jnp.int32)])
            v1 = plsc.load_gather(rows_v, [l, jnp.full((16,), 4 + c, jnp.int32)])
            v2 = plsc.load_gather(rows_v, [l, jnp.full((16,), 8 + c, jnp.int32)])
            oc = b0 * v0 + b1 * v1 + b2 * v2
            oc = jnp.where(cov, oc, zero16f)
            plsc.store_scatter(pt_v, [l3 + c], oc)
        pltpu.sync_copy(pt_v, point_hbm.at[pl.ds(3 * base, 3 * CP)])
        pltpu.sync_copy(mask_v, mask_hbm.at[pl.ds(base, CP)])

  return pl.kernel(
      body,
      out_type=(
          jax.ShapeDtypeStruct((3 * N,), jnp.float32),
          jax.ShapeDtypeStruct((N,), jnp.int32),
      ),
      mesh=_mesh(),
      compiler_params=_params,
      scratch_types=[
          pltpu.VMEM((CP,), jnp.int32),
          pltpu.VMEM((CP,), jnp.int32),
          pltpu.VMEM((CP, 16), jnp.float32),
          pltpu.VMEM((3 * CP,), jnp.float32),
          pltpu.VMEM((3 * CP,), jnp.float32),
          pltpu.VMEM((CP,), jnp.int32),
          pltpu.SemaphoreType.DMA,
      ],
  )(fattr16, pix, bary_flat)


def kernel(vertices, faces, pix_to_face, bary_coords):
  vpad8 = jnp.pad(vertices.reshape(V, 3), ((0, 0), (0, 5)))
  faces3 = jnp.pad(faces, ((0, F_PAD - F), (0, 0)))
  fattr16 = _stage1(vpad8, faces3.reshape(-1))
  point_flat, mask_i32 = _stage2(
      fattr16, pix_to_face.reshape(-1), bary_coords.reshape(-1))
  point = point_flat.reshape(B, H, W, 3)
  mask = mask_i32.reshape(B, H, W).astype(bool)
  return point, mask


# speedup vs baseline: 15.5408x; 1.0745x over previous
import jax
import jax.numpy as jnp
from jax import lax
from jax.experimental import pallas as pl
from jax.experimental.pallas import tpu as pltpu
from jax.experimental.pallas import tpu_sc as plsc

B, H, W, V, F = 1, 1080, 1920, 100000, 200000
N = H * W
NC, NS = 2, 16
NW = NC * NS

F_PAD = 200704
CF = 32
S1_CHUNKS = F_PAD // (NW * CF)

CP = 128
TOTAL_CHUNKS = N // CP
S2_CHUNKS = -(-TOTAL_CHUNKS // NW)

_params = pltpu.CompilerParams(
    use_tc_tiling_on_sc=False, needs_layout_passes=False)


def _mesh():
  return plsc.VectorSubcoreMesh(core_axis_name="c", subcore_axis_name="s")


def _worker_id():
  return lax.axis_index("s") * NC + lax.axis_index("c")


def _stage1(vpad8, faces3_flat):
  def body(vpad_hbm, fidx_hbm, fattr_hbm,
           idx_v, vrows_v, out_v, sem_idx, sem_rows):
    wid = _worker_id()
    lane = lax.iota(jnp.int32, 16)

    def start_idx(g, p):
      pltpu.async_copy(
          fidx_hbm.at[pl.ds((wid * S1_CHUNKS + g) * (3 * CF), 3 * CF)],
          idx_v.at[p], sem_idx.at[p])

    def wait_idx(g, p):
      pltpu.make_async_copy(
          fidx_hbm.at[pl.ds((wid * S1_CHUNKS + g) * (3 * CF), 3 * CF)],
          idx_v.at[p], sem_idx.at[p]).wait()

    def start_rows(p):
      pltpu.async_copy(vpad_hbm.at[idx_v.at[p]], vrows_v.at[p],
                       sem_rows.at[p])

    def wait_rows(p):
      pltpu.make_async_copy(vpad_hbm.at[idx_v.at[p]], vrows_v.at[p],
                            sem_rows.at[p]).wait()

    start_idx(0, 0)
    wait_idx(0, 0)
    start_rows(0)
    start_idx(1, 1)

    @pl.loop(0, S1_CHUNKS)
    def _chunk(g):
      p = lax.rem(g, 2)
      q = 1 - p

      @pl.when(g + 1 < S1_CHUNKS)
      def _():
        wait_idx(g + 1, q)
        start_rows(q)

      wait_rows(p)
      for i in range(CF // 16):
        l = lane + (i * 16)
        l3 = l * 3
        for k in range(3):
          row = l3 + k
          for c in range(3):
            val = plsc.load_gather(vrows_v.at[p], [row, jnp.full((16,), c, jnp.int32)])
            plsc.store_scatter(out_v, [l, jnp.full((16,), 4 * k + c, jnp.int32)], val)
      pltpu.sync_copy(out_v, fattr_hbm.at[pl.ds((wid * S1_CHUNKS + g) * CF, CF)])

      @pl.when(g + 2 < S1_CHUNKS)
      def _():
        start_idx(g + 2, p)

  return pl.kernel(
      body,
      out_type=jax.ShapeDtypeStruct((F_PAD, 16), jnp.float32),
      mesh=_mesh(),
      compiler_params=_params,
      scratch_types=[
          pltpu.VMEM((2, 3 * CF), jnp.int32),
          pltpu.VMEM((2, 3 * CF, 8), jnp.float32),
          pltpu.VMEM((CF, 16), jnp.float32),
          pltpu.SemaphoreType.DMA((2,)),
          pltpu.SemaphoreType.DMA((2,)),
      ],
  )(vpad8, faces3_flat)


def _stage2(fattr16, pix, bary_flat):
  def body(fattr_hbm, pix_hbm, bary_hbm, point_hbm, mask_hbm,
           pidx_v, gidx_v, rows_v, bary_v, pt_v, mask_v,
           sem_pix, sem_rows, sem_bary):
    wid = _worker_id()
    lane = lax.iota(jnp.int32, 16)
    zero16f = jnp.zeros((16,), jnp.float32)
    one16i = jnp.full((16,), 1, jnp.int32)
    zero16i = jnp.zeros((16,), jnp.int32)

    def chunk_of(g):
      return g * NW + wid

    def start_pix(t, p):
      pltpu.async_copy(pix_hbm.at[pl.ds(t * CP, CP)], pidx_v.at[p],
                       sem_pix.at[p])

    def wait_pix(t, p):
      pltpu.make_async_copy(pix_hbm.at[pl.ds(t * CP, CP)], pidx_v.at[p],
                            sem_pix.at[p]).wait()

    def start_bary(t, p):
      pltpu.async_copy(bary_hbm.at[pl.ds(t * 3 * CP, 3 * CP)], bary_v.at[p],
                       sem_bary.at[p])

    def wait_bary(t, p):
      pltpu.make_async_copy(bary_hbm.at[pl.ds(t * 3 * CP, 3 * CP)],
                            bary_v.at[p], sem_bary.at[p]).wait()

    def clamp(p):
      for i in range(CP // 16):
        f = pidx_v[p, pl.ds(i * 16, 16)]
        cov = f >= 0
        gidx_v[p, pl.ds(i * 16, 16)] = jnp.maximum(f, 0)
        mask_v[p, pl.ds(i * 16, 16)] = jnp.where(cov, one16i, zero16i)

    def start_rows(p):
      pltpu.async_copy(fattr_hbm.at[gidx_v.at[p]], rows_v.at[p],
                       sem_rows.at[p])

    def wait_rows(p):
      pltpu.make_async_copy(fattr_hbm.at[gidx_v.at[p]], rows_v.at[p],
                            sem_rows.at[p]).wait()

    start_pix(chunk_of(0), 0)
    wait_pix(chunk_of(0), 0)
    clamp(0)
    start_rows(0)
    start_bary(chunk_of(0), 0)
    start_pix(chunk_of(1), 1)

    @pl.loop(0, S2_CHUNKS)
    def _chunk(g):
      p = lax.rem(g, 2)
      q = 1 - p
      t = chunk_of(g)

      @pl.when(chunk_of(g + 1) < TOTAL_CHUNKS)
      def _():
        wait_pix(chunk_of(g + 1), q)
        clamp(q)
        start_rows(q)
        start_bary(chunk_of(g + 1), q)

      @pl.when(t < TOTAL_CHUNKS)
      def _():
        wait_rows(p)
        wait_bary(t, p)
        for i in range(CP // 16):
          l = lane + (i * 16)
          l3 = l * 3
          cov = pidx_v[p, pl.ds(i * 16, 16)] >= 0
          b0 = plsc.load_gather(bary_v.at[p], [l3])
          b1 = plsc.load_gather(bary_v.at[p], [l3 + 1])
          b2 = plsc.load_gather(bary_v.at[p], [l3 + 2])
          for c in range(3):
            v0 = plsc.load_gather(rows_v.at[p], [l, jnp.full((16,), c, jnp.int32)])
            v1 = plsc.load_gather(rows_v.at[p], [l, jnp.full((16,), 4 + c, jnp.int32)])
            v2 = plsc.load_gather(rows_v.at[p], [l, jnp.full((16,), 8 + c, jnp.int32)])
            oc = b0 * v0 + b1 * v1 + b2 * v2
            oc = jnp.where(cov, oc, zero16f)
            plsc.store_scatter(pt_v, [l3 + c], oc)
        pltpu.sync_copy(pt_v, point_hbm.at[pl.ds(3 * t * CP, 3 * CP)])
        pltpu.sync_copy(mask_v.at[p], mask_hbm.at[pl.ds(t * CP, CP)])

      @pl.when(chunk_of(g + 2) < TOTAL_CHUNKS)
      def _():
        start_pix(chunk_of(g + 2), p)

  return pl.kernel(
      body,
      out_type=(
          jax.ShapeDtypeStruct((3 * N,), jnp.float32),
          jax.ShapeDtypeStruct((N,), jnp.int32),
      ),
      mesh=_mesh(),
      compiler_params=_params,
      scratch_types=[
          pltpu.VMEM((2, CP), jnp.int32),
          pltpu.VMEM((2, CP), jnp.int32),
          pltpu.VMEM((2, CP, 16), jnp.float32),
          pltpu.VMEM((2, 3 * CP), jnp.float32),
          pltpu.VMEM((3 * CP,), jnp.float32),
          pltpu.VMEM((2, CP), jnp.int32),
          pltpu.SemaphoreType.DMA((2,)),
          pltpu.SemaphoreType.DMA((2,)),
          pltpu.SemaphoreType.DMA((2,)),
      ],
  )(fattr16, pix, bary_flat)


def kernel(vertices, faces, pix_to_face, bary_coords):
  vpad8 = jnp.pad(vertices.reshape(V, 3), ((0, 0), (0, 5)))
  faces3 = jnp.pad(faces, ((0, F_PAD - F), (0, 0)))
  fattr16 = _stage1(vpad8, faces3.reshape(-1))
  point_flat, mask_i32 = _stage2(
      fattr16, pix_to_face.reshape(-1), bary_coords.reshape(-1))
  point = point_flat.reshape(B, H, W, 3)
  mask = mask_i32.reshape(B, H, W).astype(bool)
  return point, mask


# speedup vs baseline: 65.4626x; 4.2123x over previous
import jax
import jax.numpy as jnp
from jax import lax
from jax.experimental import pallas as pl
from jax.experimental.pallas import tpu as pltpu
from jax.experimental.pallas import tpu_sc as plsc

B, H, W, V, F = 1, 1080, 1920, 100000, 200000
N = H * W
NC, NS = 2, 16
NW = NC * NS

F_PAD = 200704
CF = 32
S1_CHUNKS = F_PAD // (NW * CF)

CP = 128
TOTAL_CHUNKS = N // CP
S2_CHUNKS = -(-TOTAL_CHUNKS // NW)

_params = pltpu.CompilerParams(
    use_tc_tiling_on_sc=False, needs_layout_passes=False)


def _mesh():
  return plsc.VectorSubcoreMesh(core_axis_name="c", subcore_axis_name="s")


def _worker_id():
  return lax.axis_index("s") * NC + lax.axis_index("c")


def _stage1(vpad8, faces3_flat):
  def body(vpad_hbm, fidx_hbm, fattr_hbm,
           idx_v, vrows_v, out_v, sem_idx, sem_rows):
    wid = _worker_id()
    lane = lax.iota(jnp.int32, 16)

    def start_idx(g, p):
      pltpu.async_copy(
          fidx_hbm.at[pl.ds((wid * S1_CHUNKS + g) * (3 * CF), 3 * CF)],
          idx_v.at[p], sem_idx.at[p])

    def wait_idx(g, p):
      pltpu.make_async_copy(
          fidx_hbm.at[pl.ds((wid * S1_CHUNKS + g) * (3 * CF), 3 * CF)],
          idx_v.at[p], sem_idx.at[p]).wait()

    def start_rows(p):
      pltpu.async_copy(vpad_hbm.at[idx_v.at[p]], vrows_v.at[p],
                       sem_rows.at[p])

    def wait_rows(p):
      pltpu.make_async_copy(vpad_hbm.at[idx_v.at[p]], vrows_v.at[p],
                            sem_rows.at[p]).wait()

    start_idx(0, 0)
    wait_idx(0, 0)
    start_rows(0)
    start_idx(1, 1)

    @pl.loop(0, S1_CHUNKS)
    def _chunk(g):
      p = lax.rem(g, 2)
      q = 1 - p

      @pl.when(g + 1 < S1_CHUNKS)
      def _():
        wait_idx(g + 1, q)
        start_rows(q)

      wait_rows(p)
      for i in range(CF // 16):
        l = lane + (i * 16)
        l3 = l * 3
        for k in range(3):
          row = l3 + k
          for c in range(3):
            val = plsc.load_gather(vrows_v.at[p], [row, jnp.full((16,), c, jnp.int32)])
            plsc.store_scatter(out_v, [l, jnp.full((16,), 4 * k + c, jnp.int32)], val)
      pltpu.sync_copy(out_v, fattr_hbm.at[pl.ds((wid * S1_CHUNKS + g) * CF, CF)])

      @pl.when(g + 2 < S1_CHUNKS)
      def _():
        start_idx(g + 2, p)

  return pl.kernel(
      body,
      out_type=jax.ShapeDtypeStruct((F_PAD, 16), jnp.float32),
      mesh=_mesh(),
      compiler_params=_params,
      scratch_types=[
          pltpu.VMEM((2, 3 * CF), jnp.int32),
          pltpu.VMEM((2, 3 * CF, 8), jnp.float32),
          pltpu.VMEM((CF, 16), jnp.float32),
          pltpu.SemaphoreType.DMA((2,)),
          pltpu.SemaphoreType.DMA((2,)),
      ],
  )(vpad8, faces3_flat)


def _stage2(fattr16, pix, bary_planes):
  def body(fattr_hbm, pix_hbm, bary_hbm, point_hbm, mask_hbm,
           pidx_v, gidx_v, rows_v, bary_v, pt_v, mask_v,
           sem_pix, sem_rows, sem_bary):
    wid = _worker_id()
    lane = lax.iota(jnp.int32, 16)
    zero16f = jnp.zeros((16,), jnp.float32)
    one16i = jnp.full((16,), 1, jnp.int32)
    zero16i = jnp.zeros((16,), jnp.int32)

    def chunk_of(g):
      return g * NW + wid

    def start_pix(t, p):
      pltpu.async_copy(pix_hbm.at[pl.ds(t * CP, CP)], pidx_v.at[p],
                       sem_pix.at[p])

    def wait_pix(t, p):
      pltpu.make_async_copy(pix_hbm.at[pl.ds(t * CP, CP)], pidx_v.at[p],
                            sem_pix.at[p]).wait()

    def start_bary(t, p):
      for k in range(3):
        pltpu.async_copy(bary_hbm.at[k].at[pl.ds(t * CP, CP)],
                         bary_v.at[p].at[k], sem_bary.at[p])

    def wait_bary(t, p):
      for k in range(3):
        pltpu.make_async_copy(bary_hbm.at[k].at[pl.ds(t * CP, CP)],
                              bary_v.at[p].at[k], sem_bary.at[p]).wait()

    def clamp(p):
      for i in range(CP // 16):
        f = pidx_v[p, pl.ds(i * 16, 16)]
        cov = f >= 0
        gidx_v[p, pl.ds(i * 16, 16)] = jnp.maximum(f, 0)
        mask_v[p, pl.ds(i * 16, 16)] = jnp.where(cov, one16i, zero16i)

    def start_rows(p):
      pltpu.async_copy(fattr_hbm.at[gidx_v.at[p]], rows_v.at[p],
                       sem_rows.at[p])

    def wait_rows(p):
      pltpu.make_async_copy(fattr_hbm.at[gidx_v.at[p]], rows_v.at[p],
                            sem_rows.at[p]).wait()

    start_pix(chunk_of(0), 0)
    wait_pix(chunk_of(0), 0)
    clamp(0)
    start_rows(0)
    start_bary(chunk_of(0), 0)
    start_pix(chunk_of(1), 1)

    @pl.loop(0, S2_CHUNKS)
    def _chunk(g):
      p = lax.rem(g, 2)
      q = 1 - p
      t = chunk_of(g)

      @pl.when(chunk_of(g + 1) < TOTAL_CHUNKS)
      def _():
        wait_pix(chunk_of(g + 1), q)
        clamp(q)
        start_rows(q)
        start_bary(chunk_of(g + 1), q)

      @pl.when(t < TOTAL_CHUNKS)
      def _():
        wait_rows(p)
        wait_bary(t, p)
        for i in range(CP // 16):
          o = i * 16
          l = lane + o
          cov = pidx_v[p, pl.ds(o, 16)] >= 0
          b0 = bary_v[p, 0, pl.ds(o, 16)]
          b1 = bary_v[p, 1, pl.ds(o, 16)]
          b2 = bary_v[p, 2, pl.ds(o, 16)]
          for c in range(3):
            v0 = plsc.load_gather(rows_v.at[p], [l, jnp.full((16,), c, jnp.int32)])
            v1 = plsc.load_gather(rows_v.at[p], [l, jnp.full((16,), 4 + c, jnp.int32)])
            v2 = plsc.load_gather(rows_v.at[p], [l, jnp.full((16,), 8 + c, jnp.int32)])
            oc = b0 * v0 + b1 * v1 + b2 * v2
            oc = jnp.where(cov, oc, zero16f)
            pt_v[c, pl.ds(o, 16)] = oc
        for c in range(3):
          pltpu.sync_copy(pt_v.at[c], point_hbm.at[c].at[pl.ds(t * CP, CP)])
        pltpu.sync_copy(mask_v.at[p], mask_hbm.at[pl.ds(t * CP, CP)])

      @pl.when(chunk_of(g + 2) < TOTAL_CHUNKS)
      def _():
        start_pix(chunk_of(g + 2), p)

  return pl.kernel(
      body,
      out_type=(
          jax.ShapeDtypeStruct((3, N), jnp.float32),
          jax.ShapeDtypeStruct((N,), jnp.int32),
      ),
      mesh=_mesh(),
      compiler_params=_params,
      scratch_types=[
          pltpu.VMEM((2, CP), jnp.int32),
          pltpu.VMEM((2, CP), jnp.int32),
          pltpu.VMEM((2, CP, 16), jnp.float32),
          pltpu.VMEM((2, 3, CP), jnp.float32),
          pltpu.VMEM((3, CP), jnp.float32),
          pltpu.VMEM((2, CP), jnp.int32),
          pltpu.SemaphoreType.DMA((2,)),
          pltpu.SemaphoreType.DMA((2,)),
          pltpu.SemaphoreType.DMA((2,)),
      ],
  )(fattr16, pix, bary_planes)


def kernel(vertices, faces, pix_to_face, bary_coords):
  vpad8 = jnp.pad(vertices.reshape(V, 3), ((0, 0), (0, 5)))
  faces3 = jnp.pad(faces, ((0, F_PAD - F), (0, 0)))
  fattr16 = _stage1(vpad8, faces3.reshape(-1))
  bary_planes = jnp.moveaxis(bary_coords.reshape(N, 3), 1, 0)
  point_planes, mask_i32 = _stage2(
      fattr16, pix_to_face.reshape(-1), bary_planes)
  point = jnp.moveaxis(point_planes, 0, 1).reshape(B, H, W, 3)
  mask = mask_i32.reshape(B, H, W).astype(bool)
  return point, mask


# speedup vs baseline: 77.5413x; 1.1845x over previous
import jax
import jax.numpy as jnp
from jax import lax
from jax.experimental import pallas as pl
from jax.experimental.pallas import tpu as pltpu
from jax.experimental.pallas import tpu_sc as plsc

B, H, W, V, F = 1, 1080, 1920, 100000, 200000
N = H * W
NC, NS = 2, 16
NW = NC * NS

F_PAD = 200704
CF = 32
S1_CHUNKS = F_PAD // (NW * CF)

GS = 128
NSUB = 4
CP = GS * NSUB
TOTAL_CHUNKS = N // CP
S2_CHUNKS = -(-TOTAL_CHUNKS // NW)

_params = pltpu.CompilerParams(
    use_tc_tiling_on_sc=False, needs_layout_passes=False)


def _mesh():
  return plsc.VectorSubcoreMesh(core_axis_name="c", subcore_axis_name="s")


def _worker_id():
  return lax.axis_index("s") * NC + lax.axis_index("c")


def _stage1(vpad8, faces3_flat):
  def body(vpad_hbm, fidx_hbm, fattr_hbm,
           idx_v, vrows_v, out_v, sem_idx, sem_rows):
    wid = _worker_id()
    lane = lax.iota(jnp.int32, 16)

    def start_idx(g, p):
      pltpu.async_copy(
          fidx_hbm.at[pl.ds((wid * S1_CHUNKS + g) * (3 * CF), 3 * CF)],
          idx_v.at[p], sem_idx.at[p])

    def wait_idx(g, p):
      pltpu.make_async_copy(
          fidx_hbm.at[pl.ds((wid * S1_CHUNKS + g) * (3 * CF), 3 * CF)],
          idx_v.at[p], sem_idx.at[p]).wait()

    def start_rows(p):
      pltpu.async_copy(vpad_hbm.at[idx_v.at[p]], vrows_v.at[p],
                       sem_rows.at[p])

    def wait_rows(p):
      pltpu.make_async_copy(vpad_hbm.at[idx_v.at[p]], vrows_v.at[p],
                            sem_rows.at[p]).wait()

    start_idx(0, 0)
    wait_idx(0, 0)
    start_rows(0)
    start_idx(1, 1)

    @pl.loop(0, S1_CHUNKS)
    def _chunk(g):
      p = lax.rem(g, 2)
      q = 1 - p

      @pl.when(g + 1 < S1_CHUNKS)
      def _():
        wait_idx(g + 1, q)
        start_rows(q)

      wait_rows(p)
      for i in range(CF // 16):
        l = lane + (i * 16)
        l3 = l * 3
        for k in range(3):
          row = l3 + k
          for c in range(3):
            val = plsc.load_gather(vrows_v.at[p], [row, jnp.full((16,), c, jnp.int32)])
            plsc.store_scatter(out_v, [l, jnp.full((16,), 4 * k + c, jnp.int32)], val)
      pltpu.sync_copy(out_v, fattr_hbm.at[pl.ds((wid * S1_CHUNKS + g) * CF, CF)])

      @pl.when(g + 2 < S1_CHUNKS)
      def _():
        start_idx(g + 2, p)

  return pl.kernel(
      body,
      out_type=jax.ShapeDtypeStruct((F_PAD, 16), jnp.float32),
      mesh=_mesh(),
      compiler_params=_params,
      scratch_types=[
          pltpu.VMEM((2, 3 * CF), jnp.int32),
          pltpu.VMEM((2, 3 * CF, 8), jnp.float32),
          pltpu.VMEM((CF, 16), jnp.float32),
          pltpu.SemaphoreType.DMA((2,)),
          pltpu.SemaphoreType.DMA((2,)),
      ],
  )(vpad8, faces3_flat)


def _stage2(fattr16, gidx2d, bary_planes):
  def body(fattr_hbm, gidx_hbm, bary_hbm, point_hbm,
           gidx_v, rows_v, bary_v, pt_v,
           sem_gidx, sem_rows, sem_bary, sem_pt):
    wid = _worker_id()
    lane = lax.iota(jnp.int32, 16)

    def chunk_of(g):
      return g * NW + wid

    def start_gidx(t, p):
      pltpu.async_copy(gidx_hbm.at[pl.ds(t * NSUB, NSUB)], gidx_v.at[p],
                       sem_gidx.at[p])

    def wait_gidx(t, p):
      pltpu.make_async_copy(gidx_hbm.at[pl.ds(t * NSUB, NSUB)], gidx_v.at[p],
                            sem_gidx.at[p]).wait()

    def start_bary(t, p):
      for k in range(3):
        pltpu.async_copy(bary_hbm.at[k].at[pl.ds(t * CP, CP)],
                         bary_v.at[p].at[k], sem_bary.at[p])

    def wait_bary(t, p):
      for k in range(3):
        pltpu.make_async_copy(bary_hbm.at[k].at[pl.ds(t * CP, CP)],
                              bary_v.at[p].at[k], sem_bary.at[p]).wait()

    def start_rows(p):
      for j in range(NSUB):
        pltpu.async_copy(fattr_hbm.at[gidx_v.at[p].at[j]],
                         rows_v.at[p].at[j], sem_rows.at[p])

    def wait_rows(p):
      for j in range(NSUB):
        pltpu.make_async_copy(fattr_hbm.at[gidx_v.at[p].at[j]],
                              rows_v.at[p].at[j], sem_rows.at[p]).wait()

    def start_pt(t, p):
      for c in range(3):
        pltpu.async_copy(pt_v.at[p].at[c], point_hbm.at[c].at[pl.ds(t * CP, CP)],
                         sem_pt.at[p])

    def wait_pt(t, p):
      for c in range(3):
        pltpu.make_async_copy(pt_v.at[p].at[c],
                              point_hbm.at[c].at[pl.ds(t * CP, CP)],
                              sem_pt.at[p]).wait()

    start_gidx(chunk_of(0), 0)
    wait_gidx(chunk_of(0), 0)
    start_rows(0)
    start_bary(chunk_of(0), 0)
    start_gidx(chunk_of(1), 1)

    @pl.loop(0, S2_CHUNKS)
    def _chunk(g):
      p = lax.rem(g, 2)
      q = 1 - p
      t = chunk_of(g)

      @pl.when(chunk_of(g + 1) < TOTAL_CHUNKS)
      def _():
        wait_gidx(chunk_of(g + 1), q)
        start_rows(q)
        start_bary(chunk_of(g + 1), q)

      @pl.when(t < TOTAL_CHUNKS)
      def _():
        wait_rows(p)
        wait_bary(t, p)

        @pl.when(g >= 2)
        def _():
          wait_pt(chunk_of(g - 2), p)

        for jsub in range(NSUB):
          for i in range(GS // 16):
            o = jsub * GS + i * 16
            r = lane + (i * 16)
            b0 = bary_v[p, 0, pl.ds(o, 16)]
            b1 = bary_v[p, 1, pl.ds(o, 16)]
            b2 = bary_v[p, 2, pl.ds(o, 16)]
            for c in range(3):
              cc = jnp.full((16,), c, jnp.int32)
              v0 = plsc.load_gather(rows_v.at[p].at[jsub], [r, cc])
              v1 = plsc.load_gather(rows_v.at[p].at[jsub], [r, cc + 4])
              v2 = plsc.load_gather(rows_v.at[p].at[jsub], [r, cc + 8])
              pt_v[p, c, pl.ds(o, 16)] = b0 * v0 + b1 * v1 + b2 * v2
        start_pt(t, p)

      @pl.when(chunk_of(g + 2) < TOTAL_CHUNKS)
      def _():
        start_gidx(chunk_of(g + 2), p)

    for dg in (S2_CHUNKS - 2, S2_CHUNKS - 1):
      @pl.when(chunk_of(dg) < TOTAL_CHUNKS)
      def _(dg=dg):
        wait_pt(chunk_of(dg), dg % 2)

  return pl.kernel(
      body,
      out_type=jax.ShapeDtypeStruct((3, N), jnp.float32),
      mesh=_mesh(),
      compiler_params=_params,
      scratch_types=[
          pltpu.VMEM((2, NSUB, GS), jnp.int32),
          pltpu.VMEM((2, NSUB, GS, 16), jnp.float32),
          pltpu.VMEM((2, 3, CP), jnp.float32),
          pltpu.VMEM((2, 3, CP), jnp.float32),
          pltpu.SemaphoreType.DMA((2,)),
          pltpu.SemaphoreType.DMA((2,)),
          pltpu.SemaphoreType.DMA((2,)),
          pltpu.SemaphoreType.DMA((2,)),
      ],
  )(fattr16, gidx2d, bary_planes)


def kernel(vertices, faces, pix_to_face, bary_coords):
  vpad8 = jnp.pad(vertices.reshape(V, 3), ((0, 1), (0, 5)))
  faces3 = jnp.pad(faces, ((0, F_PAD - F), (0, 0)),
                   constant_values=V)
  fattr16 = _stage1(vpad8, faces3.reshape(-1))
  pix = pix_to_face.reshape(N)
  gidx2d = jnp.where(pix < 0, F, pix).reshape(N // GS, GS)
  bary_planes = jnp.moveaxis(bary_coords.reshape(N, 3), 1, 0)
  point_planes = _stage2(fattr16, gidx2d, bary_planes)
  point = jnp.moveaxis(point_planes, 0, 1).reshape(B, H, W, 3)
  mask = pix_to_face != -1
  return point, mask
